# Initial kernel scaffold; baseline (speedup 1.0000x reference)
import jax
import jax.numpy as jnp
from jax.experimental import pallas as pl


def kernel(x, table):
    raise NotImplementedError("write your pallas kernel here")


# speedup vs baseline: 1.1608x; 1.1608x over previous
import functools
import math

import jax
import jax.numpy as jnp
from jax import lax
from jax.experimental import pallas as pl
from jax.experimental.pallas import tpu as pltpu
from jax.experimental.pallas import tpu_sc as plsc

VOCAB = 30522
D = 512
B_TOTAL = 4 * 8192
NC, NS, LANES = 2, 16, 16
NW = NC * NS
B_PER_W = B_TOTAL // NW
CHUNK = 64
NCHUNK = B_PER_W // CHUNK
SCALE = math.sqrt(float(D))


def _embed_body(table_hbm, idx_hbm, out_hbm, idx_v, rows_v, in_sem, out_sem):
    wid = lax.axis_index("s") * NC + lax.axis_index("c")
    base = wid * B_PER_W
    pltpu.sync_copy(idx_hbm.at[pl.ds(base, B_PER_W)], idx_v)

    def chunk_step(c, carry):
        del carry
        pltpu.async_copy(
            table_hbm.at[idx_v.at[pl.ds(c * CHUNK, CHUNK)]], rows_v, in_sem
        ).wait()

        def scale_row(r, carry2):
            del carry2
            for j in range(D // LANES):
                sl = pl.ds(j * LANES, LANES)
                rows_v[r, sl] = rows_v[r, sl] * SCALE
            return 0

        lax.fori_loop(0, CHUNK, scale_row, 0)
        pltpu.async_copy(
            rows_v, out_hbm.at[pl.ds(base + c * CHUNK, CHUNK)], out_sem
        ).wait()
        return 0

    lax.fori_loop(0, NCHUNK, chunk_step, 0)


@jax.jit
def _embed(x_flat, table):
    mesh = plsc.VectorSubcoreMesh(core_axis_name="c", subcore_axis_name="s")
    out = pl.kernel(
        _embed_body,
        out_type=jax.ShapeDtypeStruct((B_TOTAL, D), jnp.float32),
        mesh=mesh,
        scratch_types=[
            pltpu.VMEM((B_PER_W,), jnp.int32),
            pltpu.VMEM((CHUNK, D), jnp.float32),
            pltpu.SemaphoreType.DMA,
            pltpu.SemaphoreType.DMA,
        ],
    )(table, x_flat)
    return out


def kernel(x, table):
    x_flat = x.reshape(B_TOTAL).astype(jnp.int32)
    out = _embed(x_flat, table)
    return out.reshape(x.shape[0], x.shape[1], D)


# speedup vs baseline: 1.5085x; 1.2996x over previous
import math

import jax
import jax.numpy as jnp
from jax import lax
from jax.experimental import pallas as pl
from jax.experimental.pallas import tpu as pltpu
from jax.experimental.pallas import tpu_sc as plsc

VOCAB = 30522
D = 512
B_TOTAL = 4 * 8192
NC, NS, LANES = 2, 16, 16
NW = NC * NS
B_PER_W = B_TOTAL // NW
CHUNK = 64
NCHUNK = B_PER_W // CHUNK
SCALE = math.sqrt(float(D))


def _scale_chunk(buf):
    def scale_row(r, carry):
        del carry
        for j in range(D // LANES):
            sl = pl.ds(j * LANES, LANES)
            buf[r, sl] = buf[r, sl] * SCALE
        return 0

    lax.fori_loop(0, CHUNK, scale_row, 0)


def _embed_body(table_hbm, idx_hbm, out_hbm, idx_v, rows0, rows1,
                in_sem0, in_sem1, out_sem0, out_sem1):
    wid = lax.axis_index("s") * NC + lax.axis_index("c")
    base = wid * B_PER_W
    pltpu.sync_copy(idx_hbm.at[pl.ds(base, B_PER_W)], idx_v)

    bufs = (rows0, rows1)
    in_sems = (in_sem0, in_sem1)
    out_sems = (out_sem0, out_sem1)

    def start_gather(c):
        b = c % 2
        return pltpu.async_copy(
            table_hbm.at[idx_v.at[pl.ds(c * CHUNK, CHUNK)]], bufs[b], in_sems[b]
        )

    def start_out(c):
        b = c % 2
        return pltpu.async_copy(
            bufs[b], out_hbm.at[pl.ds(base + c * CHUNK, CHUNK)], out_sems[b]
        )

    gathers = [None] * NCHUNK
    outs = [None] * NCHUNK
    gathers[0] = start_gather(0)
    for c in range(NCHUNK):
        b = c % 2
        if c + 1 < NCHUNK:
            if c >= 1:
                outs[c - 1].wait()
            gathers[c + 1] = start_gather(c + 1)
        gathers[c].wait()
        _scale_chunk(bufs[b])
        outs[c] = start_out(c)
    outs[NCHUNK - 2].wait()
    outs[NCHUNK - 1].wait()


@jax.jit
def _embed(x_flat, table):
    mesh = plsc.VectorSubcoreMesh(core_axis_name="c", subcore_axis_name="s")
    out = pl.kernel(
        _embed_body,
        out_type=jax.ShapeDtypeStruct((B_TOTAL, D), jnp.float32),
        mesh=mesh,
        scratch_types=[
            pltpu.VMEM((B_PER_W,), jnp.int32),
            pltpu.VMEM((CHUNK, D), jnp.float32),
            pltpu.VMEM((CHUNK, D), jnp.float32),
            pltpu.SemaphoreType.DMA,
            pltpu.SemaphoreType.DMA,
            pltpu.SemaphoreType.DMA,
            pltpu.SemaphoreType.DMA,
        ],
    )(table, x_flat)
    return out


def kernel(x, table):
    x_flat = x.reshape(B_TOTAL).astype(jnp.int32)
    out = _embed(x_flat, table)
    return out.reshape(x.shape[0], x.shape[1], D)
